# baseline (speedup 1.0000x reference)
import jax
import jax.numpy as jnp
from jax.experimental import pallas as pl
from jax.experimental.pallas import tpu as pltpu

_DN = (((1,), (1,)), ((), ()))
_NCHUNK = 8


def _body(x_ref, adj_ref, w1_ref, b1_ref, w2_ref, b2_ref, fc1w_hbm,
          fc1b_ref, fc2w_ref, fc2b_ref, fc3w_ref, fc3b_ref, out_ref,
          wbuf, flat_s, sems):
    nout, kdim = wbuf.shape
    rows = nout // _NCHUNK
    for k in range(_NCHUNK):
        sl = pl.ds(k * rows, rows)
        pltpu.make_async_copy(fc1w_hbm.at[sl, :], wbuf.at[sl, :],
                              sems.at[k]).start()

    s1 = jnp.dot(x_ref[...], w1_ref[...], preferred_element_type=jnp.float32)
    h1 = jax.nn.relu(
        jnp.dot(adj_ref[...], s1, preferred_element_type=jnp.float32)
        + b1_ref[...].reshape(1, -1)
    )
    s2 = jnp.dot(h1, w2_ref[...], preferred_element_type=jnp.float32)
    h2 = jax.nn.relu(
        jnp.dot(adj_ref[...], s2, preferred_element_type=jnp.float32)
        + b2_ref[...].reshape(1, -1)
    )
    n, nclass = h2.shape
    for r in range(n):
        flat_s[0:1, r * nclass:(r + 1) * nclass] = h2[r:r + 1, :]

    accs = []
    for k in range(_NCHUNK):
        sl = pl.ds(k * rows, rows)
        pltpu.make_async_copy(fc1w_hbm.at[sl, :], wbuf.at[sl, :],
                              sems.at[k]).wait()
        t = wbuf[k * rows:(k + 1) * rows, :] * flat_s[0:1, :]
        accr = jnp.zeros((rows, 128), jnp.float32)
        for g in range(kdim // 128):
            accr = accr + t[:, g * 128:(g + 1) * 128]
        accs.append(accr)
    acc = jnp.concatenate(accs, axis=0)

    a1 = jax.nn.relu(acc.sum(axis=1).reshape(1, nout)
                     + fc1b_ref[...].reshape(1, -1))
    a2 = jax.nn.relu(
        jax.lax.dot_general(a1, fc2w_ref[...], _DN,
                            preferred_element_type=jnp.float32)
        + fc2b_ref[...].reshape(1, -1)
    )
    a3 = (jnp.sum(a2 * fc3w_ref[...], axis=1, keepdims=True)
          + fc3b_ref[...].reshape(1, -1))
    out_ref[...] = jax.nn.sigmoid(a3).reshape(1)


def kernel(x, adj, W1, b1, W2, b2, fc1_w, fc1_b, fc2_w, fc2_b, fc3_w, fc3_b):
    nout, kdim = fc1_w.shape
    vmem = pl.BlockSpec(memory_space=pltpu.MemorySpace.VMEM)

    y = pl.pallas_call(
        _body,
        in_specs=[vmem, vmem, vmem, vmem, vmem, vmem,
                  pl.BlockSpec(memory_space=pl.ANY),
                  vmem, vmem, vmem, vmem, vmem],
        out_shape=jax.ShapeDtypeStruct((1,), jnp.float32),
        scratch_shapes=[
            pltpu.VMEM((nout, kdim), jnp.float32),
            pltpu.VMEM((1, kdim), jnp.float32),
            pltpu.SemaphoreType.DMA((_NCHUNK,)),
        ],
    )(x, adj, W1, b1, W2, b2, fc1_w, fc1_b, fc2_w, fc2_b, fc3_w, fc3_b)

    return y
